# baseline (speedup 1.0000x reference)
import functools

import jax
import jax.numpy as jnp
from jax import lax
from jax.experimental import pallas as pl
from jax.experimental.pallas import tpu as pltpu
from jax.experimental.pallas import tpu_sc as plsc

N = 10000
D_IN = 128
D_OUT = 128
R = 8
NB = 4

NC = 2
NS = 16
NW = NC * NS
CHUNK = 312
TAIL = N - NW * CHUNK
L = 16


def _sc_segment_partials(x, edge_type, edge_norm):
    mesh = plsc.VectorSubcoreMesh(core_axis_name="c", subcore_axis_name="s")

    n_groups = CHUNK // L + 1

    def _acc_group(accf, xflat, tvec, nvec, row0):
        for j in range(L):
            t = tvec[j]
            w = nvec[j]
            xbase = (row0 + j) * D_IN
            abase = t * D_IN
            for s in range(D_IN // L):
                plsc.addupdate(accf.at[pl.ds(abase + s * L, L)],
                               w * xflat[pl.ds(xbase + s * L, L)])

    @functools.partial(
        pl.kernel,
        mesh=mesh,
        out_type=jax.ShapeDtypeStruct((NW, R * D_IN), jnp.float32),
        scratch_types=[
            pltpu.VMEM((CHUNK * D_IN,), jnp.float32),
            pltpu.VMEM((CHUNK,), jnp.int32),
            pltpu.VMEM((CHUNK,), jnp.float32),
            pltpu.VMEM((R * D_IN,), jnp.float32),
            pltpu.VMEM((TAIL * D_IN,), jnp.float32),
            pltpu.VMEM((TAIL,), jnp.int32),
            pltpu.VMEM((TAIL,), jnp.float32),
            pltpu.SemaphoreType.DMA,
            pltpu.SemaphoreType.DMA,
            pltpu.SemaphoreType.DMA,
            pltpu.SemaphoreType.DMA,
        ],
        compiler_params=pltpu.CompilerParams(needs_layout_passes=False),
    )
    def k(x_hbm, et_hbm, en_hbm, out_hbm, xv, tv, nv, acc, xt, tt, nt,
          sem_x1, sem_x2, sem_t, sem_n):
        wid = lax.axis_index("s") * NC + lax.axis_index("c")
        base = wid * CHUNK
        half = (n_groups // 2) * L
        hx1 = pltpu.async_copy(x_hbm.at[pl.ds(base * D_IN, half * D_IN)],
                               xv.at[pl.ds(0, half * D_IN)], sem_x1)
        hx2 = pltpu.async_copy(
            x_hbm.at[pl.ds((base + half) * D_IN, (CHUNK - half) * D_IN)],
            xv.at[pl.ds(half * D_IN, (CHUNK - half) * D_IN)], sem_x2)
        ht = pltpu.async_copy(et_hbm.at[pl.ds(base, CHUNK)], tv, sem_t)
        hn = pltpu.async_copy(en_hbm.at[pl.ds(base, CHUNK)], nv, sem_n)

        zero = jnp.zeros((L,), jnp.float32)
        for s in range(R * D_IN // L):
            acc[pl.ds(s * L, L)] = zero
        ht.wait()
        hn.wait()

        lanes = lax.iota(jnp.int32, L)

        def body(g):
            is_last = g == (n_groups - 1)
            row0 = jnp.where(is_last, CHUNK - L, g * L)
            lane_lo = jnp.where(is_last, L - (CHUNK % L), 0)
            tvec = tv[pl.ds(row0, L)]
            nvec = jnp.where(lanes >= lane_lo, nv[pl.ds(row0, L)], 0.0)
            _acc_group(acc, xv, tvec, nvec, row0)

        hx1.wait()
        plsc.parallel_loop(0, n_groups // 2, unroll=2)(body)
        hx2.wait()
        plsc.parallel_loop(n_groups // 2, n_groups, unroll=2)(body)

        @pl.when(wid == 0)
        def _tail():
            tbase = NW * CHUNK
            pltpu.sync_copy(x_hbm.at[pl.ds(tbase * D_IN, TAIL * D_IN)], xt)
            pltpu.sync_copy(et_hbm.at[pl.ds(tbase, TAIL)], tt)
            pltpu.sync_copy(en_hbm.at[pl.ds(tbase, TAIL)], nt)
            _acc_group(acc, xt, tt[pl.ds(0, L)], nt[pl.ds(0, L)], 0)

        pltpu.sync_copy(acc, out_hbm.at[wid])

    return k(x.reshape(N * D_IN), edge_type, edge_norm)


BLK = 5000


def _tc_kernel(part_ref, att_ref, basis_ref, x_ref, w_ref, b_ref, o_ref, sref):
    @pl.when(pl.program_id(0) == 0)
    def _():
        basis_o = jnp.sum(basis_ref[...], axis=2)
        s = jax.lax.dot(att_ref[...], basis_o,
                        preferred_element_type=jnp.float32)
        t = jnp.sum(part_ref[...], axis=0)
        sref[0] = jnp.sum(t * s) * (1.0 / (N * D_OUT))

    o_ref[...] = (
        jnp.dot(x_ref[...], w_ref[...], preferred_element_type=jnp.float32)
        + sref[0]
        + b_ref[...]
    )


def _tc_out(partials, att, basis, x, weight, bias2d):
    return pl.pallas_call(
        _tc_kernel,
        grid=(N // BLK,),
        in_specs=[
            pl.BlockSpec((NW, R, D_IN), lambda i: (0, 0, 0)),
            pl.BlockSpec((R, NB), lambda i: (0, 0)),
            pl.BlockSpec((NB, D_IN, D_OUT), lambda i: (0, 0, 0)),
            pl.BlockSpec((BLK, D_IN), lambda i: (i, 0)),
            pl.BlockSpec((D_IN, D_OUT), lambda i: (0, 0)),
            pl.BlockSpec((1, D_OUT), lambda i: (0, 0)),
        ],
        out_specs=pl.BlockSpec((BLK, D_OUT), lambda i: (i, 0)),
        out_shape=jax.ShapeDtypeStruct((N, D_OUT), jnp.float32),
        scratch_shapes=[pltpu.SMEM((1,), jnp.float32)],
    )(partials, att, basis, x, weight, bias2d)


def kernel(x, edge_index, edge_type, edge_norm, basis, att, weight, bias):
    del edge_index
    partials = _sc_segment_partials(x, edge_type, edge_norm)
    partials = partials.reshape(NW, R, D_IN)
    return _tc_out(partials, att, basis, x, weight, bias.reshape(1, D_OUT))


# speedup vs baseline: 1.1200x; 1.1200x over previous
import functools

import jax
import jax.numpy as jnp
from jax import lax
from jax.experimental import pallas as pl
from jax.experimental.pallas import tpu as pltpu
from jax.experimental.pallas import tpu_sc as plsc

N = 10000
D_IN = 128
D_OUT = 128
R = 8
NB = 4

NC = 2
NS = 16
NW = NC * NS
CHUNK = 312
TAIL = N - NW * CHUNK
L = 16


def _sc_segment_partials(x, edge_type, edge_norm):
    mesh = plsc.VectorSubcoreMesh(core_axis_name="c", subcore_axis_name="s")

    n_groups = CHUNK // L + 1

    def _acc_group(accf, xflat, tvec, nvec, row0):
        for j in range(L):
            t = tvec[j]
            w = nvec[j]
            xbase = (row0 + j) * D_IN
            abase = t * D_IN
            for s in range(D_IN // L):
                plsc.addupdate(accf.at[pl.ds(abase + s * L, L)],
                               w * xflat[pl.ds(xbase + s * L, L)])

    @functools.partial(
        pl.kernel,
        mesh=mesh,
        out_type=jax.ShapeDtypeStruct((NW, R * D_IN), jnp.float32),
        scratch_types=[
            pltpu.VMEM((CHUNK * D_IN,), jnp.float32),
            pltpu.VMEM((CHUNK,), jnp.int32),
            pltpu.VMEM((CHUNK,), jnp.float32),
            pltpu.VMEM((R * D_IN,), jnp.float32),
            pltpu.VMEM((TAIL * D_IN,), jnp.float32),
            pltpu.VMEM((TAIL,), jnp.int32),
            pltpu.VMEM((TAIL,), jnp.float32),
            pltpu.SemaphoreType.DMA,
            pltpu.SemaphoreType.DMA,
            pltpu.SemaphoreType.DMA,
        ],
        compiler_params=pltpu.CompilerParams(needs_layout_passes=False),
    )
    def k(x_hbm, et_hbm, en_hbm, out_hbm, xv, tv, nv, acc, xt, tt, nt,
          sem_x1, sem_t, sem_n):
        wid = lax.axis_index("s") * NC + lax.axis_index("c")
        base = wid * CHUNK
        hx = pltpu.async_copy(x_hbm.at[pl.ds(base * D_IN, CHUNK * D_IN)],
                              xv, sem_x1)
        ht = pltpu.async_copy(et_hbm.at[pl.ds(base, CHUNK)], tv, sem_t)
        hn = pltpu.async_copy(en_hbm.at[pl.ds(base, CHUNK)], nv, sem_n)

        zero = jnp.zeros((L,), jnp.float32)
        for s in range(R * D_IN // L):
            acc[pl.ds(s * L, L)] = zero
        ht.wait()
        hn.wait()
        hx.wait()

        lanes = lax.iota(jnp.int32, L)

        @plsc.parallel_loop(0, n_groups, unroll=2)
        def body(g):
            is_last = g == (n_groups - 1)
            row0 = jnp.where(is_last, CHUNK - L, g * L)
            lane_lo = jnp.where(is_last, L - (CHUNK % L), 0)
            tvec = tv[pl.ds(row0, L)]
            nvec = jnp.where(lanes >= lane_lo, nv[pl.ds(row0, L)], 0.0)
            _acc_group(acc, xv, tvec, nvec, row0)

        @pl.when(wid == 0)
        def _tail():
            tbase = NW * CHUNK
            pltpu.sync_copy(x_hbm.at[pl.ds(tbase * D_IN, TAIL * D_IN)], xt)
            pltpu.sync_copy(et_hbm.at[pl.ds(tbase, TAIL)], tt)
            pltpu.sync_copy(en_hbm.at[pl.ds(tbase, TAIL)], nt)
            _acc_group(acc, xt, tt[pl.ds(0, L)], nt[pl.ds(0, L)], 0)

        pltpu.sync_copy(acc, out_hbm.at[wid])

    return k(x.reshape(N * D_IN), edge_type, edge_norm)


BLK = 5000


def _tc_kernel(part_ref, att_ref, basis_ref, x_ref, w_ref, b_ref, o_ref, sref):
    @pl.when(pl.program_id(0) == 0)
    def _():
        basis_o = jnp.sum(basis_ref[...], axis=2)
        s = jax.lax.dot(att_ref[...], basis_o,
                        preferred_element_type=jnp.float32)
        t = jnp.sum(part_ref[...], axis=0)
        sref[0] = jnp.sum(t * s) * (1.0 / (N * D_OUT))

    o_ref[...] = (
        jnp.dot(x_ref[...], w_ref[...], preferred_element_type=jnp.float32)
        + sref[0]
        + b_ref[...]
    )


def _tc_out(partials, att, basis, x, weight, bias2d):
    return pl.pallas_call(
        _tc_kernel,
        grid=(N // BLK,),
        in_specs=[
            pl.BlockSpec((NW, R, D_IN), lambda i: (0, 0, 0)),
            pl.BlockSpec((R, NB), lambda i: (0, 0)),
            pl.BlockSpec((NB, D_IN, D_OUT), lambda i: (0, 0, 0)),
            pl.BlockSpec((BLK, D_IN), lambda i: (i, 0)),
            pl.BlockSpec((D_IN, D_OUT), lambda i: (0, 0)),
            pl.BlockSpec((1, D_OUT), lambda i: (0, 0)),
        ],
        out_specs=pl.BlockSpec((BLK, D_OUT), lambda i: (i, 0)),
        out_shape=jax.ShapeDtypeStruct((N, D_OUT), jnp.float32),
        scratch_shapes=[pltpu.SMEM((1,), jnp.float32)],
    )(partials, att, basis, x, weight, bias2d)


def kernel(x, edge_index, edge_type, edge_norm, basis, att, weight, bias):
    del edge_index
    partials = _sc_segment_partials(x, edge_type, edge_norm)
    partials = partials.reshape(NW, R, D_IN)
    return _tc_out(partials, att, basis, x, weight, bias.reshape(1, D_OUT))
